# baseline (device time: 11940 ns/iter reference)
import jax
import jax.numpy as jnp
from jax import lax
from jax.experimental import pallas as pl
from jax.experimental.pallas import tpu as pltpu

C = 5
CK = 32
FWD = C * CK
OVL = 512 - 2 * FWD


def kernel(x):
    m, n = x.shape
    assert m == 2 * FWD + OVL

    def body(x_ref, out_ref, xv, vstage, vrecv, yf_send, yf_recv,
             ov_sems, f_send, f_recv, ldma_sems, in_sems):
        my_x = lax.axis_index("x")
        my_y = lax.axis_index("y")
        my_z = lax.axis_index("z")
        s = (my_x + my_z) % 2
        ynbr = (my_x, 1 - my_y, my_z)
        xnbr = (1 - my_x, my_y, my_z)

        fwd_off = s * (FWD + OVL)
        rfwd_off = (1 - s) * (FWD + OVL)
        my_base = my_y * m
        rem_base = (1 - my_y) * m

        dma_fwd = pltpu.make_async_copy(
            x_ref.at[pl.ds(fwd_off, FWD)], xv.at[pl.ds(fwd_off, FWD)],
            in_sems.at[0],
        )
        dma_fwd.start()
        dma_ovl = pltpu.make_async_copy(
            x_ref.at[pl.ds(FWD, OVL)], xv.at[pl.ds(FWD, OVL)],
            in_sems.at[1],
        )
        dma_ovl.start()
        dma_rf = pltpu.make_async_copy(
            x_ref.at[pl.ds(rfwd_off, FWD)], xv.at[pl.ds(rfwd_off, FWD)],
            in_sems.at[2],
        )
        dma_rf.start()

        barrier = pltpu.get_barrier_semaphore()
        for nbr in (ynbr, xnbr):
            pl.semaphore_signal(
                barrier, inc=1, device_id=nbr,
                device_id_type=pl.DeviceIdType.MESH,
            )
        pl.semaphore_wait(barrier, 2)

        dma_fwd.wait()
        vstage[pl.ds(fwd_off, FWD), :] = (
            xv[pl.ds(fwd_off, FWD), :].astype(jnp.bfloat16)
        )
        y_rdmas = []
        for c in range(C):
            r = pltpu.make_async_remote_copy(
                src_ref=vstage.at[pl.ds(fwd_off + c * CK, CK)],
                dst_ref=vrecv.at[pl.ds(c * CK, CK)],
                send_sem=yf_send.at[c],
                recv_sem=yf_recv.at[c],
                device_id=ynbr,
                device_id_type=pl.DeviceIdType.MESH,
            )
            r.start()
            y_rdmas.append(r)

        dma_ovl.wait()
        vstage[pl.ds(FWD, OVL), :] = (
            xv[pl.ds(FWD, OVL), :].astype(jnp.bfloat16)
        )
        ov = pltpu.make_async_remote_copy(
            src_ref=vstage.at[pl.ds(FWD, OVL)],
            dst_ref=out_ref.at[pl.ds(my_base + FWD, OVL)],
            send_sem=ov_sems.at[0],
            recv_sem=ov_sems.at[1],
            device_id=ynbr,
            device_id_type=pl.DeviceIdType.MESH,
        )
        ov.start()

        dma_rf.wait()
        vstage[pl.ds(rfwd_off, FWD), :] = (
            xv[pl.ds(rfwd_off, FWD), :].astype(jnp.bfloat16)
        )
        own = pltpu.make_async_copy(
            vstage, out_ref.at[pl.ds(my_base, m)], ldma_sems.at[0]
        )
        own.start()

        f_rdmas = []
        for c in range(C):
            recv = pltpu.make_async_remote_copy(
                src_ref=vstage.at[pl.ds(fwd_off + c * CK, CK)],
                dst_ref=vrecv.at[pl.ds(c * CK, CK)],
                send_sem=yf_send.at[c],
                recv_sem=yf_recv.at[c],
                device_id=ynbr,
                device_id_type=pl.DeviceIdType.MESH,
            )
            recv.wait_recv()
            f = pltpu.make_async_remote_copy(
                src_ref=vrecv.at[pl.ds(c * CK, CK)],
                dst_ref=out_ref.at[pl.ds(rem_base + fwd_off + c * CK, CK)],
                send_sem=f_send.at[c],
                recv_sem=f_recv.at[c],
                device_id=xnbr,
                device_id_type=pl.DeviceIdType.MESH,
            )
            f.start()
            f_rdmas.append(f)

        stg = pltpu.make_async_copy(
            vrecv, out_ref.at[pl.ds(rem_base + fwd_off, FWD)],
            ldma_sems.at[1],
        )
        stg.start()

        ov_in = pltpu.make_async_remote_copy(
            src_ref=vstage.at[pl.ds(FWD, OVL)],
            dst_ref=out_ref.at[pl.ds(rem_base + FWD, OVL)],
            send_sem=ov_sems.at[0],
            recv_sem=ov_sems.at[1],
            device_id=ynbr,
            device_id_type=pl.DeviceIdType.MESH,
        )
        ov_in.wait_recv()

        for c in range(C):
            rin = pltpu.make_async_remote_copy(
                src_ref=vrecv.at[pl.ds(c * CK, CK)],
                dst_ref=out_ref.at[pl.ds(rem_base + rfwd_off + c * CK, CK)],
                send_sem=f_send.at[c],
                recv_sem=f_recv.at[c],
                device_id=xnbr,
                device_id_type=pl.DeviceIdType.MESH,
            )
            rin.wait_recv()

        for r in y_rdmas:
            r.wait_send()
        ov.wait_send()
        for r in f_rdmas:
            r.wait_send()
        own.wait()
        stg.wait()

    return pl.pallas_call(
        body,
        out_shape=jax.ShapeDtypeStruct((2 * m, n), jnp.bfloat16),
        in_specs=[pl.BlockSpec(memory_space=pltpu.MemorySpace.HBM)],
        out_specs=pl.BlockSpec(memory_space=pl.ANY),
        scratch_shapes=[
            pltpu.VMEM((m, n), jnp.float32),
            pltpu.VMEM((m, n), jnp.bfloat16),
            pltpu.VMEM((FWD, n), jnp.bfloat16),
            pltpu.SemaphoreType.DMA((C,)),
            pltpu.SemaphoreType.DMA((C,)),
            pltpu.SemaphoreType.DMA((2,)),
            pltpu.SemaphoreType.DMA((C,)),
            pltpu.SemaphoreType.DMA((C,)),
            pltpu.SemaphoreType.DMA((2,)),
            pltpu.SemaphoreType.DMA((3,)),
        ],
        compiler_params=pltpu.CompilerParams(collective_id=0),
    )(x)


# device time: 11650 ns/iter; 1.0249x vs baseline; 1.0249x over previous
import jax
import jax.numpy as jnp
from jax import lax
from jax.experimental import pallas as pl
from jax.experimental.pallas import tpu as pltpu

C = 5
CK = 32
FWD = C * CK
OVL = 512 - 2 * FWD


def kernel(x):
    m, n = x.shape
    assert m == 2 * FWD + OVL

    def body(x_ref, out_ref, vstage, vrecv, yf_send, yf_recv, ov_sems,
             f_send, f_recv, ldma_sems):
        my_x = lax.axis_index("x")
        my_y = lax.axis_index("y")
        my_z = lax.axis_index("z")
        s = (my_x + my_z) % 2
        ynbr = (my_x, 1 - my_y, my_z)
        xnbr = (1 - my_x, my_y, my_z)

        fwd_off = s * (FWD + OVL)
        rfwd_off = (1 - s) * (FWD + OVL)
        my_base = my_y * m
        rem_base = (1 - my_y) * m

        vstage[pl.ds(fwd_off, FWD), :] = (
            x_ref[pl.ds(fwd_off, FWD), :].astype(jnp.bfloat16)
        )
        vstage[pl.ds(FWD, OVL), :] = (
            x_ref[pl.ds(FWD, OVL), :].astype(jnp.bfloat16)
        )
        vstage[pl.ds(rfwd_off, FWD), :] = (
            x_ref[pl.ds(rfwd_off, FWD), :].astype(jnp.bfloat16)
        )
        own = pltpu.make_async_copy(
            vstage, out_ref.at[pl.ds(my_base, m)], ldma_sems.at[0]
        )
        own.start()

        barrier = pltpu.get_barrier_semaphore()
        for nbr in (ynbr, xnbr):
            pl.semaphore_signal(
                barrier, inc=1, device_id=nbr,
                device_id_type=pl.DeviceIdType.MESH,
            )
        pl.semaphore_wait(barrier, 2)

        y_rdmas = []
        for c in range(C):
            r = pltpu.make_async_remote_copy(
                src_ref=vstage.at[pl.ds(fwd_off + c * CK, CK)],
                dst_ref=vrecv.at[pl.ds(c * CK, CK)],
                send_sem=yf_send.at[c],
                recv_sem=yf_recv.at[c],
                device_id=ynbr,
                device_id_type=pl.DeviceIdType.MESH,
            )
            r.start()
            y_rdmas.append(r)

        ov = pltpu.make_async_remote_copy(
            src_ref=vstage.at[pl.ds(FWD, OVL)],
            dst_ref=out_ref.at[pl.ds(my_base + FWD, OVL)],
            send_sem=ov_sems.at[0],
            recv_sem=ov_sems.at[1],
            device_id=ynbr,
            device_id_type=pl.DeviceIdType.MESH,
        )
        ov.start()

        f_rdmas = []
        for c in range(C):
            recv = pltpu.make_async_remote_copy(
                src_ref=vstage.at[pl.ds(fwd_off + c * CK, CK)],
                dst_ref=vrecv.at[pl.ds(c * CK, CK)],
                send_sem=yf_send.at[c],
                recv_sem=yf_recv.at[c],
                device_id=ynbr,
                device_id_type=pl.DeviceIdType.MESH,
            )
            recv.wait_recv()
            f = pltpu.make_async_remote_copy(
                src_ref=vrecv.at[pl.ds(c * CK, CK)],
                dst_ref=out_ref.at[pl.ds(rem_base + fwd_off + c * CK, CK)],
                send_sem=f_send.at[c],
                recv_sem=f_recv.at[c],
                device_id=xnbr,
                device_id_type=pl.DeviceIdType.MESH,
            )
            f.start()
            f_rdmas.append(f)

        stg = pltpu.make_async_copy(
            vrecv, out_ref.at[pl.ds(rem_base + fwd_off, FWD)],
            ldma_sems.at[1],
        )
        stg.start()

        ov_in = pltpu.make_async_remote_copy(
            src_ref=vstage.at[pl.ds(FWD, OVL)],
            dst_ref=out_ref.at[pl.ds(rem_base + FWD, OVL)],
            send_sem=ov_sems.at[0],
            recv_sem=ov_sems.at[1],
            device_id=ynbr,
            device_id_type=pl.DeviceIdType.MESH,
        )
        ov_in.wait_recv()

        for c in range(C):
            rin = pltpu.make_async_remote_copy(
                src_ref=vrecv.at[pl.ds(c * CK, CK)],
                dst_ref=out_ref.at[pl.ds(rem_base + rfwd_off + c * CK, CK)],
                send_sem=f_send.at[c],
                recv_sem=f_recv.at[c],
                device_id=xnbr,
                device_id_type=pl.DeviceIdType.MESH,
            )
            rin.wait_recv()

        for r in y_rdmas:
            r.wait_send()
        ov.wait_send()
        for r in f_rdmas:
            r.wait_send()
        own.wait()
        stg.wait()

    return pl.pallas_call(
        body,
        out_shape=jax.ShapeDtypeStruct((2 * m, n), jnp.bfloat16),
        in_specs=[pl.BlockSpec(memory_space=pltpu.VMEM)],
        out_specs=pl.BlockSpec(memory_space=pl.ANY),
        scratch_shapes=[
            pltpu.VMEM((m, n), jnp.bfloat16),
            pltpu.VMEM((FWD, n), jnp.bfloat16),
            pltpu.SemaphoreType.DMA((C,)),
            pltpu.SemaphoreType.DMA((C,)),
            pltpu.SemaphoreType.DMA((2,)),
            pltpu.SemaphoreType.DMA((C,)),
            pltpu.SemaphoreType.DMA((C,)),
            pltpu.SemaphoreType.DMA((2,)),
        ],
        compiler_params=pltpu.CompilerParams(collective_id=0),
    )(x)


# device time: 11524 ns/iter; 1.0361x vs baseline; 1.0109x over previous
import jax
import jax.numpy as jnp
from jax import lax
from jax.experimental import pallas as pl
from jax.experimental.pallas import tpu as pltpu

C = 5
CK = 32
FWD = C * CK
OVL = 512 - 2 * FWD


def kernel(x):
    m, n = x.shape
    assert m == 2 * FWD + OVL

    def body(x_ref, out_ref, vstage, vrecv, yf_send, yf_recv, ov_sems,
             f_send, f_recv, ldma_sems):
        my_x = lax.axis_index("x")
        my_y = lax.axis_index("y")
        my_z = lax.axis_index("z")
        s = (my_x + my_z) % 2
        ynbr = (my_x, 1 - my_y, my_z)
        xnbr = (1 - my_x, my_y, my_z)

        fwd_off = s * (FWD + OVL)
        rfwd_off = (1 - s) * (FWD + OVL)
        my_base = my_y * m
        rem_base = (1 - my_y) * m

        barrier = pltpu.get_barrier_semaphore()
        for nbr in (ynbr, xnbr):
            pl.semaphore_signal(
                barrier, inc=1, device_id=nbr,
                device_id_type=pl.DeviceIdType.MESH,
            )

        vstage[pl.ds(fwd_off, FWD), :] = (
            x_ref[pl.ds(fwd_off, FWD), :].astype(jnp.bfloat16)
        )
        vstage[pl.ds(FWD, OVL), :] = (
            x_ref[pl.ds(FWD, OVL), :].astype(jnp.bfloat16)
        )
        vstage[pl.ds(rfwd_off, FWD), :] = (
            x_ref[pl.ds(rfwd_off, FWD), :].astype(jnp.bfloat16)
        )
        own = pltpu.make_async_copy(
            vstage, out_ref.at[pl.ds(my_base, m)], ldma_sems.at[0]
        )
        own.start()

        pl.semaphore_wait(barrier, 2)

        y_rdmas = []
        for c in range(C):
            r = pltpu.make_async_remote_copy(
                src_ref=vstage.at[pl.ds(fwd_off + c * CK, CK)],
                dst_ref=vrecv.at[pl.ds(c * CK, CK)],
                send_sem=yf_send.at[c],
                recv_sem=yf_recv.at[c],
                device_id=ynbr,
                device_id_type=pl.DeviceIdType.MESH,
            )
            r.start()
            y_rdmas.append(r)

        ov = pltpu.make_async_remote_copy(
            src_ref=vstage.at[pl.ds(FWD, OVL)],
            dst_ref=out_ref.at[pl.ds(my_base + FWD, OVL)],
            send_sem=ov_sems.at[0],
            recv_sem=ov_sems.at[1],
            device_id=ynbr,
            device_id_type=pl.DeviceIdType.MESH,
        )
        ov.start()

        f_rdmas = []
        for c in range(C):
            recv = pltpu.make_async_remote_copy(
                src_ref=vstage.at[pl.ds(fwd_off + c * CK, CK)],
                dst_ref=vrecv.at[pl.ds(c * CK, CK)],
                send_sem=yf_send.at[c],
                recv_sem=yf_recv.at[c],
                device_id=ynbr,
                device_id_type=pl.DeviceIdType.MESH,
            )
            recv.wait_recv()
            f = pltpu.make_async_remote_copy(
                src_ref=vrecv.at[pl.ds(c * CK, CK)],
                dst_ref=out_ref.at[pl.ds(rem_base + fwd_off + c * CK, CK)],
                send_sem=f_send.at[c],
                recv_sem=f_recv.at[c],
                device_id=xnbr,
                device_id_type=pl.DeviceIdType.MESH,
            )
            f.start()
            f_rdmas.append(f)

        stg = pltpu.make_async_copy(
            vrecv, out_ref.at[pl.ds(rem_base + fwd_off, FWD)],
            ldma_sems.at[1],
        )
        stg.start()

        ov_in = pltpu.make_async_remote_copy(
            src_ref=vstage.at[pl.ds(FWD, OVL)],
            dst_ref=out_ref.at[pl.ds(rem_base + FWD, OVL)],
            send_sem=ov_sems.at[0],
            recv_sem=ov_sems.at[1],
            device_id=ynbr,
            device_id_type=pl.DeviceIdType.MESH,
        )
        ov_in.wait_recv()

        for c in range(C):
            rin = pltpu.make_async_remote_copy(
                src_ref=vrecv.at[pl.ds(c * CK, CK)],
                dst_ref=out_ref.at[pl.ds(rem_base + rfwd_off + c * CK, CK)],
                send_sem=f_send.at[c],
                recv_sem=f_recv.at[c],
                device_id=xnbr,
                device_id_type=pl.DeviceIdType.MESH,
            )
            rin.wait_recv()

        for r in y_rdmas:
            r.wait_send()
        ov.wait_send()
        for r in f_rdmas:
            r.wait_send()
        own.wait()
        stg.wait()

    return pl.pallas_call(
        body,
        out_shape=jax.ShapeDtypeStruct((2 * m, n), jnp.bfloat16),
        in_specs=[pl.BlockSpec(memory_space=pltpu.VMEM)],
        out_specs=pl.BlockSpec(memory_space=pl.ANY),
        scratch_shapes=[
            pltpu.VMEM((m, n), jnp.bfloat16),
            pltpu.VMEM((FWD, n), jnp.bfloat16),
            pltpu.SemaphoreType.DMA((C,)),
            pltpu.SemaphoreType.DMA((C,)),
            pltpu.SemaphoreType.DMA((2,)),
            pltpu.SemaphoreType.DMA((C,)),
            pltpu.SemaphoreType.DMA((C,)),
            pltpu.SemaphoreType.DMA((2,)),
        ],
        compiler_params=pltpu.CompilerParams(collective_id=0),
    )(x)
